# baseline (speedup 1.0000x reference)
import functools

import jax
import jax.numpy as jnp
from jax import lax
from jax.experimental import pallas as pl
from jax.experimental.pallas import tpu as pltpu
from jax.experimental.pallas import tpu_sc as plsc

_FACT = 128
_HIDDEN = 2048
_B = 8192
_C = 4
_BC = _B // _C

_NC, _NS = 2, 16
_NW = _NC * _NS
_B_PER_W = _BC // _NW


def _gather_body(table_hbm, idx_hbm, out_hbm, idx_v, rows_v, sem):
    wid = lax.axis_index("s") * _NC + lax.axis_index("c")
    base = wid * _B_PER_W
    pltpu.sync_copy(idx_hbm.at[pl.ds(base, _B_PER_W)], idx_v)
    pltpu.async_copy(table_hbm.at[idx_v], rows_v, sem).wait()
    pltpu.sync_copy(rows_v, out_hbm.at[pl.ds(base, _B_PER_W)])


_sc_gather = functools.partial(
    pl.kernel,
    out_type=jax.ShapeDtypeStruct((_BC, _FACT), jnp.float32),
    mesh=plsc.VectorSubcoreMesh(core_axis_name="c", subcore_axis_name="s"),
    scratch_types=[
        pltpu.VMEM((_B_PER_W,), jnp.int32),
        pltpu.VMEM((_B_PER_W, _FACT), jnp.float32),
        pltpu.SemaphoreType.DMA,
    ],
)(_gather_body)


_BM = 1024
_MB_PER_C = _BC // _BM


def _mm_first_body(x_ref, w_ref, o_ref):
    o_ref[...] = lax.dot_general(
        x_ref[...],
        w_ref[...],
        dimension_numbers=(((1,), (1,)), ((), ())),
        preferred_element_type=jnp.float32,
    )


def _mm_chain_body(x_ref, w_ref, yprev_ref, o_ref):
    del yprev_ref
    o_ref[...] = lax.dot_general(
        x_ref[...],
        w_ref[...],
        dimension_numbers=(((1,), (1,)), ((), ())),
        preferred_element_type=jnp.float32,
    )


def _make_mm(c):
    out_spec = pl.BlockSpec((_BM, _HIDDEN), lambda i, c=c: (i + c * _MB_PER_C, 0))
    x_spec = pl.BlockSpec((_BM, _FACT), lambda i: (i, 0))
    w_spec = pl.BlockSpec((_HIDDEN, _FACT), lambda i: (0, 0))
    if c == 0:
        return pl.pallas_call(
            _mm_first_body,
            grid=(_MB_PER_C,),
            in_specs=[x_spec, w_spec],
            out_specs=out_spec,
            out_shape=jax.ShapeDtypeStruct((_B, _HIDDEN), jnp.float32),
        )
    return pl.pallas_call(
        _mm_chain_body,
        grid=(_MB_PER_C,),
        in_specs=[x_spec, w_spec, pl.BlockSpec(memory_space=pl.ANY)],
        out_specs=out_spec,
        out_shape=jax.ShapeDtypeStruct((_B, _HIDDEN), jnp.float32),
        input_output_aliases={2: 0},
    )


_mms = [_make_mm(c) for c in range(_C)]


def kernel(input_ids, embed_weight, proj_weight):
    batch, seq = input_ids.shape
    ids = input_ids.reshape(-1).astype(jnp.int32)
    xs = [_sc_gather(embed_weight, ids[c * _BC : (c + 1) * _BC]) for c in range(_C)]
    y = _mms[0](xs[0], proj_weight)
    for c in range(1, _C):
        y = _mms[c](xs[c], proj_weight, y)
    return y.reshape(batch, seq, _HIDDEN)


# speedup vs baseline: 1.2019x; 1.2019x over previous
import functools

import jax
import jax.numpy as jnp
from jax import lax
from jax.experimental import pallas as pl
from jax.experimental.pallas import tpu as pltpu
from jax.experimental.pallas import tpu_sc as plsc

_FACT = 128
_HIDDEN = 2048
_B = 8192

_NC, _NS = 2, 16
_NW = _NC * _NS
_B_PER_W = _B // _NW
_CHUNK = 64
_N_CHUNKS = _B_PER_W // _CHUNK


def _gather_body(table_hbm, idx_hbm, out_hbm, idx_v, rows_v, gsem, osem):
    wid = lax.axis_index("s") * _NC + lax.axis_index("c")
    base = wid * _B_PER_W
    pltpu.sync_copy(idx_hbm.at[pl.ds(base, _B_PER_W)], idx_v)
    gathers = []
    for j in range(_N_CHUNKS):
        gathers.append(
            pltpu.async_copy(
                table_hbm.at[idx_v.at[pl.ds(j * _CHUNK, _CHUNK)]],
                rows_v.at[pl.ds(j * _CHUNK, _CHUNK)],
                gsem,
            )
        )
    writes = []
    for j in range(_N_CHUNKS):
        gathers[j].wait()
        writes.append(
            pltpu.async_copy(
                rows_v.at[pl.ds(j * _CHUNK, _CHUNK)],
                out_hbm.at[pl.ds(base + j * _CHUNK, _CHUNK)],
                osem,
            )
        )
    for cp in writes:
        cp.wait()


_sc_gather = functools.partial(
    pl.kernel,
    out_type=jax.ShapeDtypeStruct((_B, _FACT), jnp.float32),
    mesh=plsc.VectorSubcoreMesh(core_axis_name="c", subcore_axis_name="s"),
    scratch_types=[
        pltpu.VMEM((_B_PER_W,), jnp.int32),
        pltpu.VMEM((_B_PER_W, _FACT), jnp.float32),
        pltpu.SemaphoreType.DMA,
        pltpu.SemaphoreType.DMA,
    ],
)(_gather_body)


def _mm_body(x_ref, w_ref, o_ref):
    o_ref[...] = lax.dot_general(
        x_ref[...],
        w_ref[...],
        dimension_numbers=(((1,), (1,)), ((), ())),
        preferred_element_type=jnp.float32,
    )


_BM = 1024

_matmul = pl.pallas_call(
    _mm_body,
    grid=(_B // _BM,),
    in_specs=[
        pl.BlockSpec((_BM, _FACT), lambda i: (i, 0)),
        pl.BlockSpec((_HIDDEN, _FACT), lambda i: (0, 0)),
    ],
    out_specs=pl.BlockSpec((_BM, _HIDDEN), lambda i: (i, 0)),
    out_shape=jax.ShapeDtypeStruct((_B, _HIDDEN), jnp.float32),
)


def kernel(input_ids, embed_weight, proj_weight):
    batch, seq = input_ids.shape
    ids = input_ids.reshape(-1).astype(jnp.int32)
    x = _sc_gather(embed_weight, ids)
    y = _matmul(x, proj_weight)
    return y.reshape(batch, seq, _HIDDEN)
